# baseline (speedup 1.0000x reference)
import functools

import jax
import jax.numpy as jnp
from jax import lax
from jax.experimental import pallas as pl
from jax.experimental.pallas import tpu as pltpu
from jax.experimental.pallas import tpu_sc as plsc

B, C, H, W = 4, 96, 384, 384
HW = H * W
CLS = 19
CPAD = 32
NC, NS, L = 2, 16, 16
NW = NC * NS
CHUNKS_PER_B = NW // B
ROWS = H // CHUNKS_PER_B
GROUPS_PER_ROW = W // L
SLAB = C * CPAD
NSLOT = 3
SROWS = NSLOT * ROWS

_mesh = plsc.VectorSubcoreMesh(core_axis_name="c", subcore_axis_name="s")
_params = pltpu.CompilerParams(needs_layout_passes=False, use_tc_tiling_on_sc=True)


def _zero_acc(acc):
    z = jnp.zeros((L,), jnp.float32)
    for r in range(2 * L):
        acc[pl.ds(r * L, L)] = z


def _fold_acc(acc):
    lo = acc[pl.ds(0, L)]
    hi = acc[pl.ds(L, L)]
    for r in range(1, L):
        lo = lo + acc[pl.ds(r * CPAD, L)]
        hi = hi + acc[pl.ds(r * CPAD + L, L)]
    return lo, hi


@functools.partial(
    pl.kernel,
    out_type=(
        jax.ShapeDtypeStruct((NW * SLAB,), jnp.float32),
        jax.ShapeDtypeStruct((NW * CPAD,), jnp.float32),
    ),
    mesh=_mesh,
    scratch_types=[
        pltpu.VMEM((ROWS, W), jnp.int32),
        pltpu.VMEM((ROWS, W), jnp.float32),
        pltpu.VMEM((ROWS, W), jnp.float32),
        pltpu.VMEM((L * CPAD,), jnp.float32),
        pltpu.VMEM((SLAB,), jnp.float32),
        pltpu.VMEM((CPAD,), jnp.float32),
        pltpu.VMEM_SHARED((NS * SROWS, W), jnp.float32),
        pltpu.SemaphoreType.DMA,
        pltpu.SemaphoreType.DMA,
        pltpu.SemaphoreType.DMA,
        pltpu.SemaphoreType.DMA,
        pltpu.SemaphoreType.DMA,
    ],
    compiler_params=_params,
)
def _seg_sums(img_hbm, gt_hbm, sums_out, cnts_out,
              gt_v, buf0, buf1, acc, slab, cslab, spm,
              semd0, semd1, semd2, sems0, sems1):
    wid = lax.axis_index("s") * NC + lax.axis_index("c")
    b = wid // CHUNKS_PER_B
    r0 = (wid % CHUNKS_PER_B) * ROWS
    lanes32 = lax.iota(jnp.int32, L) * CPAD
    srow = lax.axis_index("s") * SROWS
    semd = [semd0, semd1, semd2]
    sems = [sems0, sems1]
    bufs = [buf0, buf1]

    def slot(k):
        return spm.at[pl.ds(srow + k * ROWS, ROWS), :]

    def d_issue(ch, k):
        pltpu.async_copy(img_hbm.at[b, ch, pl.ds(r0, ROWS), :], slot(k), semd[k])

    def d_wait(k):
        pltpu.make_async_copy(img_hbm.at[b, 0, pl.ds(r0, ROWS), :],
                              slot(k), semd[k]).wait()

    HR = ROWS // 2

    def half(k, h):
        return spm.at[pl.ds(srow + k * ROWS + h * HR, HR), :]

    def s_issue(k, v):
        pltpu.async_copy(half(k, 0), bufs[v].at[pl.ds(0, HR), :], sems[v])
        pltpu.async_copy(half(k, 1), bufs[v].at[pl.ds(HR, HR), :], sems[v])

    def s_wait(k, v):
        pltpu.make_async_copy(half(k, 0), bufs[v].at[pl.ds(0, HR), :], sems[v]).wait()
        pltpu.make_async_copy(half(k, 1), bufs[v].at[pl.ds(HR, HR), :], sems[v]).wait()

    pltpu.sync_copy(gt_hbm.at[b, pl.ds(r0, ROWS), :], gt_v)
    d_issue(0, 0)
    d_issue(1, 1)
    d_issue(2, 2)

    _zero_acc(acc)
    ones = jnp.ones((L,), jnp.float32)

    def cbody(r, carry):
        for k in range(GROUPS_PER_ROW):
            idx = lanes32 + gt_v[r, pl.ds(k * L, L)]
            plsc.addupdate_scatter(acc, [idx], ones)
        return carry

    lax.fori_loop(0, ROWS, cbody, 0)
    lo, hi = _fold_acc(acc)
    cslab[pl.ds(0, L)] = lo
    cslab[pl.ds(L, L)] = hi
    pltpu.sync_copy(cslab, cnts_out.at[pl.ds(wid * CPAD, CPAD)])
    _zero_acc(acc)

    d_wait(0)
    s_issue(0, 0)

    def process(ch, v):
        buf = bufs[v]

        def rbody(r, carry):
            for k in range(GROUPS_PER_ROW):
                idx = lanes32 + gt_v[r, pl.ds(k * L, L)]
                plsc.addupdate_scatter(acc, [idx], buf[r, pl.ds(k * L, L)])
            return carry

        lax.fori_loop(0, ROWS, rbody, 0)
        slo, shi = _fold_acc(acc)
        slab[pl.ds(ch * CPAD, L)] = slo
        slab[pl.ds(ch * CPAD + L, L)] = shi
        _zero_acc(acc)

    def qbody(q, carry):
        for j in range(6):
            ch = 6 * q + j
            sk, v = j % NSLOT, j % 2
            s_wait(sk, v)

            @pl.when(ch + 3 < C)
            def _refill():
                d_issue(ch + 3, sk)

            process(ch, v)

            @pl.when(ch + 1 < C)
            def _stage_next():
                d_wait((sk + 1) % NSLOT)
                s_issue((sk + 1) % NSLOT, 1 - v)
        return carry

    lax.fori_loop(0, C // 6, qbody, 0)
    pltpu.sync_copy(slab, sums_out.at[pl.ds(wid * SLAB, SLAB)])


@functools.partial(
    pl.kernel,
    out_type=jax.ShapeDtypeStruct((B, C, H, W), jnp.float32),
    mesh=_mesh,
    scratch_types=[
        pltpu.VMEM((ROWS, W), jnp.int32),
        pltpu.VMEM((ROWS, W), jnp.float32),
        pltpu.VMEM((ROWS, W), jnp.float32),
        pltpu.VMEM((SLAB,), jnp.float32),
        pltpu.VMEM((SLAB,), jnp.float32),
        pltpu.VMEM((CPAD,), jnp.float32),
        pltpu.VMEM((CPAD,), jnp.float32),
        pltpu.VMEM_SHARED((NS * SROWS, W), jnp.float32),
        pltpu.SemaphoreType.DMA,
        pltpu.SemaphoreType.DMA,
        pltpu.SemaphoreType.DMA,
        pltpu.SemaphoreType.DMA,
        pltpu.SemaphoreType.DMA,
        pltpu.SemaphoreType.DMA,
    ],
    compiler_params=_params,
)
def _apply(sums_hbm, cnts_hbm, gt_hbm, out_hbm,
           gt_v, buf0, buf1, mtab, tmp, cacc, ctmp, spm,
           semd0, semd1, semd2, sems0, sems1, semg):
    wid = lax.axis_index("s") * NC + lax.axis_index("c")
    b = wid // CHUNKS_PER_B
    r0 = (wid % CHUNKS_PER_B) * ROWS
    w0 = b * CHUNKS_PER_B
    srow = lax.axis_index("s") * SROWS
    semd = [semd0, semd1, semd2]
    sems = [sems0, sems1]
    bufs = [buf0, buf1]

    def slot(k):
        return spm.at[pl.ds(srow + k * ROWS, ROWS), :]

    def d_issue(ch, k):
        pltpu.async_copy(slot(k), out_hbm.at[b, ch, pl.ds(r0, ROWS), :], semd[k])

    def d_wait(k):
        pltpu.make_async_copy(slot(k), out_hbm.at[b, 0, pl.ds(r0, ROWS), :],
                              semd[k]).wait()

    HR = ROWS // 2

    def half(k, h):
        return spm.at[pl.ds(srow + k * ROWS + h * HR, HR), :]

    def s_issue(v, k):
        pltpu.async_copy(bufs[v].at[pl.ds(0, HR), :], half(k, 0), sems[v])
        pltpu.async_copy(bufs[v].at[pl.ds(HR, HR), :], half(k, 1), sems[v])

    def s_wait(v, k):
        pltpu.make_async_copy(bufs[v].at[pl.ds(0, HR), :], half(k, 0), sems[v]).wait()
        pltpu.make_async_copy(bufs[v].at[pl.ds(HR, HR), :], half(k, 1), sems[v]).wait()

    pltpu.async_copy(gt_hbm.at[b, pl.ds(r0, ROWS), :], gt_v, semg)

    pltpu.sync_copy(sums_hbm.at[pl.ds(w0 * SLAB, SLAB)], mtab)
    pltpu.sync_copy(cnts_hbm.at[pl.ds(w0 * CPAD, CPAD)], cacc)
    for j in range(1, CHUNKS_PER_B):
        pltpu.sync_copy(sums_hbm.at[pl.ds((w0 + j) * SLAB, SLAB)], tmp)
        pltpu.sync_copy(cnts_hbm.at[pl.ds((w0 + j) * CPAD, CPAD)], ctmp)
        cacc[pl.ds(0, L)] = cacc[pl.ds(0, L)] + ctmp[pl.ds(0, L)]
        cacc[pl.ds(L, L)] = cacc[pl.ds(L, L)] + ctmp[pl.ds(L, L)]

        def abody(r, carry):
            o = r * CPAD
            mtab[pl.ds(o, L)] = mtab[pl.ds(o, L)] + tmp[pl.ds(o, L)]
            mtab[pl.ds(o + L, L)] = mtab[pl.ds(o + L, L)] + tmp[pl.ds(o + L, L)]
            return carry

        lax.fori_loop(0, C, abody, 0)

    rlo = 1.0 / (cacc[pl.ds(0, L)] + 1e-8)
    rhi = 1.0 / (cacc[pl.ds(L, L)] + 1e-8)

    def mbody(r, carry):
        o = r * CPAD
        mtab[pl.ds(o, L)] = mtab[pl.ds(o, L)] * rlo
        mtab[pl.ds(o + L, L)] = mtab[pl.ds(o + L, L)] * rhi
        return carry

    lax.fori_loop(0, C, mbody, 0)

    pltpu.make_async_copy(gt_hbm.at[b, pl.ds(r0, ROWS), :], gt_v, semg).wait()

    def process(ch, v):
        buf = bufs[v]
        cv = jnp.full((L,), ch * CPAD, dtype=jnp.int32)

        def rbody(r, carry):
            for k in range(GROUPS_PER_ROW):
                g = gt_v[r, pl.ds(k * L, L)]
                buf[r, pl.ds(k * L, L)] = plsc.load_gather(mtab, [cv + g])
            return carry

        lax.fori_loop(0, ROWS, rbody, 0)

    def qbody(q, carry):
        for j in range(6):
            ch = 6 * q + j
            sk, v = j % NSLOT, j % 2

            @pl.when(ch >= NSLOT)
            def _slot_free():
                d_wait(sk)

            process(ch, v)

            s_issue(v, sk)

            @pl.when(ch >= 1)
            def _push_prev():
                s_wait(1 - v, (sk + NSLOT - 1) % NSLOT)
                d_issue(ch - 1, (sk + NSLOT - 1) % NSLOT)
        return carry

    lax.fori_loop(0, C // 6, qbody, 0)

    s_wait(1, 2)
    d_issue(C - 1, 2)
    d_wait(0)
    d_wait(1)
    d_wait(2)


def kernel(img, gt):
    gt3 = gt.reshape(B, H, W).astype(jnp.int32)
    sums, cnts = _seg_sums(img, gt3)
    return _apply(sums, cnts, gt3)


# speedup vs baseline: 1.0538x; 1.0538x over previous
import functools

import jax
import jax.numpy as jnp
from jax import lax
from jax.experimental import pallas as pl
from jax.experimental.pallas import tpu as pltpu
from jax.experimental.pallas import tpu_sc as plsc

B, C, H, W = 4, 96, 384, 384
HW = H * W
CLS = 19
CPAD = 32
NC, NS, L = 2, 16, 16
NW = NC * NS
CHUNKS_PER_B = NW // B
ROWS = H // CHUNKS_PER_B
GROUPS_PER_ROW = W // L
SLAB = C * CPAD
NSLOT = 3
SROWS = NSLOT * ROWS

_mesh = plsc.VectorSubcoreMesh(core_axis_name="c", subcore_axis_name="s")
_params = pltpu.CompilerParams(needs_layout_passes=False, use_tc_tiling_on_sc=True)


def _zero_acc(acc):
    z = jnp.zeros((L,), jnp.float32)
    for r in range(2 * L):
        acc[pl.ds(r * L, L)] = z


def _fold_acc(acc):
    lo = acc[pl.ds(0, L)]
    hi = acc[pl.ds(L, L)]
    for r in range(1, L):
        lo = lo + acc[pl.ds(r * CPAD, L)]
        hi = hi + acc[pl.ds(r * CPAD + L, L)]
    return lo, hi


@functools.partial(
    pl.kernel,
    out_type=(
        jax.ShapeDtypeStruct((NW * SLAB,), jnp.float32),
        jax.ShapeDtypeStruct((NW * CPAD,), jnp.float32),
    ),
    mesh=_mesh,
    scratch_types=[
        pltpu.VMEM((ROWS, W), jnp.int32),
        pltpu.VMEM((ROWS, W), jnp.float32),
        pltpu.VMEM((ROWS, W), jnp.float32),
        pltpu.VMEM((L * CPAD,), jnp.float32),
        pltpu.VMEM((SLAB,), jnp.float32),
        pltpu.VMEM((CPAD,), jnp.float32),
        pltpu.VMEM_SHARED((NS * SROWS, W), jnp.float32),
        pltpu.SemaphoreType.DMA,
        pltpu.SemaphoreType.DMA,
        pltpu.SemaphoreType.DMA,
        pltpu.SemaphoreType.DMA,
        pltpu.SemaphoreType.DMA,
    ],
    compiler_params=_params,
)
def _seg_sums(img_hbm, gt_hbm, sums_out, cnts_out,
              gt_v, buf0, buf1, acc, slab, cslab, spm,
              semd0, semd1, semd2, sems0, sems1):
    wid = lax.axis_index("s") * NC + lax.axis_index("c")
    b = wid // CHUNKS_PER_B
    r0 = (wid % CHUNKS_PER_B) * ROWS
    lanes = lax.iota(jnp.int32, L)
    rots = [((lanes + r) % L) * CPAD for r in range(4)]
    srow = lax.axis_index("s") * SROWS
    semd = [semd0, semd1, semd2]
    sems = [sems0, sems1]
    bufs = [buf0, buf1]

    def slot(k):
        return spm.at[pl.ds(srow + k * ROWS, ROWS), :]

    def d_issue(ch, k):
        pltpu.async_copy(img_hbm.at[b, ch, pl.ds(r0, ROWS), :], slot(k), semd[k])

    def d_wait(k):
        pltpu.make_async_copy(img_hbm.at[b, 0, pl.ds(r0, ROWS), :],
                              slot(k), semd[k]).wait()

    HR = ROWS // 2

    def half(k, h):
        return spm.at[pl.ds(srow + k * ROWS + h * HR, HR), :]

    def s_issue(k, v):
        pltpu.async_copy(half(k, 0), bufs[v].at[pl.ds(0, HR), :], sems[v])
        pltpu.async_copy(half(k, 1), bufs[v].at[pl.ds(HR, HR), :], sems[v])

    def s_wait(k, v):
        pltpu.make_async_copy(half(k, 0), bufs[v].at[pl.ds(0, HR), :], sems[v]).wait()
        pltpu.make_async_copy(half(k, 1), bufs[v].at[pl.ds(HR, HR), :], sems[v]).wait()

    pltpu.sync_copy(gt_hbm.at[b, pl.ds(r0, ROWS), :], gt_v)
    d_issue(0, 0)
    d_issue(1, 1)
    d_issue(2, 2)

    _zero_acc(acc)
    ones = jnp.ones((L,), jnp.float32)

    def cbody(r, carry):
        for k in range(GROUPS_PER_ROW):
            idx = rots[k % 4] + gt_v[r, pl.ds(k * L, L)]
            plsc.addupdate_scatter(acc, [idx], ones)
        return carry

    lax.fori_loop(0, ROWS, cbody, 0)
    lo, hi = _fold_acc(acc)
    cslab[pl.ds(0, L)] = lo
    cslab[pl.ds(L, L)] = hi
    pltpu.sync_copy(cslab, cnts_out.at[pl.ds(wid * CPAD, CPAD)])
    _zero_acc(acc)

    d_wait(0)
    s_issue(0, 0)

    def process(ch, v):
        buf = bufs[v]

        def rbody(r, carry):
            for k in range(GROUPS_PER_ROW):
                idx = rots[k % 4] + gt_v[r, pl.ds(k * L, L)]
                plsc.addupdate_scatter(acc, [idx], buf[r, pl.ds(k * L, L)])
            return carry

        lax.fori_loop(0, ROWS, rbody, 0)
        slo, shi = _fold_acc(acc)
        slab[pl.ds(ch * CPAD, L)] = slo
        slab[pl.ds(ch * CPAD + L, L)] = shi
        _zero_acc(acc)

    def qbody(q, carry):
        for j in range(6):
            ch = 6 * q + j
            sk, v = j % NSLOT, j % 2
            s_wait(sk, v)

            @pl.when(ch + 3 < C)
            def _refill():
                d_issue(ch + 3, sk)

            @pl.when(ch + 1 < C)
            def _stage_next():
                d_wait((sk + 1) % NSLOT)
                s_issue((sk + 1) % NSLOT, 1 - v)

            process(ch, v)
        return carry

    lax.fori_loop(0, C // 6, qbody, 0)
    pltpu.sync_copy(slab, sums_out.at[pl.ds(wid * SLAB, SLAB)])


@functools.partial(
    pl.kernel,
    out_type=jax.ShapeDtypeStruct((B, C, H, W), jnp.float32),
    mesh=_mesh,
    scratch_types=[
        pltpu.VMEM((ROWS, W), jnp.int32),
        pltpu.VMEM((ROWS, W), jnp.float32),
        pltpu.VMEM((ROWS, W), jnp.float32),
        pltpu.VMEM((SLAB,), jnp.float32),
        pltpu.VMEM((SLAB,), jnp.float32),
        pltpu.VMEM((CPAD,), jnp.float32),
        pltpu.VMEM((CPAD,), jnp.float32),
        pltpu.VMEM_SHARED((NS * SROWS, W), jnp.float32),
        pltpu.SemaphoreType.DMA,
        pltpu.SemaphoreType.DMA,
        pltpu.SemaphoreType.DMA,
        pltpu.SemaphoreType.DMA,
        pltpu.SemaphoreType.DMA,
        pltpu.SemaphoreType.DMA,
    ],
    compiler_params=_params,
)
def _apply(sums_hbm, cnts_hbm, gt_hbm, out_hbm,
           gt_v, buf0, buf1, mtab, tmp, cacc, ctmp, spm,
           semd0, semd1, semd2, sems0, sems1, semg):
    wid = lax.axis_index("s") * NC + lax.axis_index("c")
    b = wid // CHUNKS_PER_B
    r0 = (wid % CHUNKS_PER_B) * ROWS
    w0 = b * CHUNKS_PER_B
    srow = lax.axis_index("s") * SROWS
    semd = [semd0, semd1, semd2]
    sems = [sems0, sems1]
    bufs = [buf0, buf1]

    def slot(k):
        return spm.at[pl.ds(srow + k * ROWS, ROWS), :]

    def d_issue(ch, k):
        pltpu.async_copy(slot(k), out_hbm.at[b, ch, pl.ds(r0, ROWS), :], semd[k])

    def d_wait(k):
        pltpu.make_async_copy(slot(k), out_hbm.at[b, 0, pl.ds(r0, ROWS), :],
                              semd[k]).wait()

    HR = ROWS // 2

    def half(k, h):
        return spm.at[pl.ds(srow + k * ROWS + h * HR, HR), :]

    def s_issue(v, k):
        pltpu.async_copy(bufs[v].at[pl.ds(0, HR), :], half(k, 0), sems[v])
        pltpu.async_copy(bufs[v].at[pl.ds(HR, HR), :], half(k, 1), sems[v])

    def s_wait(v, k):
        pltpu.make_async_copy(bufs[v].at[pl.ds(0, HR), :], half(k, 0), sems[v]).wait()
        pltpu.make_async_copy(bufs[v].at[pl.ds(HR, HR), :], half(k, 1), sems[v]).wait()

    pltpu.async_copy(gt_hbm.at[b, pl.ds(r0, ROWS), :], gt_v, semg)

    pltpu.sync_copy(sums_hbm.at[pl.ds(w0 * SLAB, SLAB)], mtab)
    pltpu.sync_copy(cnts_hbm.at[pl.ds(w0 * CPAD, CPAD)], cacc)
    for j in range(1, CHUNKS_PER_B):
        pltpu.sync_copy(sums_hbm.at[pl.ds((w0 + j) * SLAB, SLAB)], tmp)
        pltpu.sync_copy(cnts_hbm.at[pl.ds((w0 + j) * CPAD, CPAD)], ctmp)
        cacc[pl.ds(0, L)] = cacc[pl.ds(0, L)] + ctmp[pl.ds(0, L)]
        cacc[pl.ds(L, L)] = cacc[pl.ds(L, L)] + ctmp[pl.ds(L, L)]

        def abody(r, carry):
            o = r * CPAD
            mtab[pl.ds(o, L)] = mtab[pl.ds(o, L)] + tmp[pl.ds(o, L)]
            mtab[pl.ds(o + L, L)] = mtab[pl.ds(o + L, L)] + tmp[pl.ds(o + L, L)]
            return carry

        lax.fori_loop(0, C, abody, 0)

    rlo = 1.0 / (cacc[pl.ds(0, L)] + 1e-8)
    rhi = 1.0 / (cacc[pl.ds(L, L)] + 1e-8)

    def mbody(r, carry):
        o = r * CPAD
        mtab[pl.ds(o, L)] = mtab[pl.ds(o, L)] * rlo
        mtab[pl.ds(o + L, L)] = mtab[pl.ds(o + L, L)] * rhi
        return carry

    lax.fori_loop(0, C, mbody, 0)

    pltpu.make_async_copy(gt_hbm.at[b, pl.ds(r0, ROWS), :], gt_v, semg).wait()

    def process(ch, v):
        buf = bufs[v]
        cv = jnp.full((L,), ch * CPAD, dtype=jnp.int32)

        def rbody(r, carry):
            for k in range(GROUPS_PER_ROW):
                g = gt_v[r, pl.ds(k * L, L)]
                buf[r, pl.ds(k * L, L)] = plsc.load_gather(mtab, [cv + g])
            return carry

        lax.fori_loop(0, ROWS, rbody, 0)

    def qbody(q, carry):
        for j in range(6):
            ch = 6 * q + j
            sk, v = j % NSLOT, j % 2

            @pl.when(ch >= NSLOT)
            def _slot_free():
                d_wait(sk)

            process(ch, v)

            s_issue(v, sk)

            @pl.when(ch >= 1)
            def _push_prev():
                s_wait(1 - v, (sk + NSLOT - 1) % NSLOT)
                d_issue(ch - 1, (sk + NSLOT - 1) % NSLOT)
        return carry

    lax.fori_loop(0, C // 6, qbody, 0)

    s_wait(1, 2)
    d_issue(C - 1, 2)
    d_wait(0)
    d_wait(1)
    d_wait(2)


def kernel(img, gt):
    gt3 = gt.reshape(B, H, W).astype(jnp.int32)
    sums, cnts = _seg_sums(img, gt3)
    return _apply(sums, cnts, gt3)


# speedup vs baseline: 2.2467x; 2.1320x over previous
import functools

import jax
import jax.numpy as jnp
from jax import lax
from jax.experimental import pallas as pl
from jax.experimental.pallas import tpu as pltpu
from jax.experimental.pallas import tpu_sc as plsc

B, C, H, W = 4, 96, 384, 384
HW = H * W
CLS = 19
CPAD = 32
NC, NS, L = 2, 16, 16
NW = NC * NS
CHUNKS_PER_B = NW // B
ROWS = H // CHUNKS_PER_B
GROUPS_PER_ROW = W // L
SLAB = C * CPAD
NSLOT = 3
SROWS = NSLOT * ROWS

_mesh = plsc.VectorSubcoreMesh(core_axis_name="c", subcore_axis_name="s")
_params = pltpu.CompilerParams(needs_layout_passes=False, use_tc_tiling_on_sc=True)


def _zero_acc(acc):
    z = jnp.zeros((L,), jnp.float32)
    for r in range(2 * L):
        acc[pl.ds(r * L, L)] = z


def _fold_acc(acc):
    lo = acc[pl.ds(0, L)]
    hi = acc[pl.ds(L, L)]
    for r in range(1, L):
        lo = lo + acc[pl.ds(r * CPAD, L)]
        hi = hi + acc[pl.ds(r * CPAD + L, L)]
    return lo, hi


@functools.partial(
    pl.kernel,
    out_type=(
        jax.ShapeDtypeStruct((NW * SLAB,), jnp.float32),
        jax.ShapeDtypeStruct((NW * CPAD,), jnp.float32),
    ),
    mesh=_mesh,
    scratch_types=[
        pltpu.VMEM((ROWS, W), jnp.int32),
        pltpu.VMEM((ROWS, W), jnp.float32),
        pltpu.VMEM((ROWS, W), jnp.float32),
        pltpu.VMEM((L * CPAD,), jnp.float32),
        pltpu.VMEM((SLAB,), jnp.float32),
        pltpu.VMEM((CPAD,), jnp.float32),
        pltpu.VMEM_SHARED((NS * SROWS, W), jnp.float32),
        pltpu.SemaphoreType.DMA,
        pltpu.SemaphoreType.DMA,
        pltpu.SemaphoreType.DMA,
        pltpu.SemaphoreType.DMA,
        pltpu.SemaphoreType.DMA,
    ],
    compiler_params=_params,
)
def _seg_sums(img_hbm, gt_hbm, sums_out, cnts_out,
              gt_v, buf0, buf1, acc, slab, cslab, spm,
              semd0, semd1, semd2, sems0, sems1):
    wid = lax.axis_index("s") * NC + lax.axis_index("c")
    b = wid // CHUNKS_PER_B
    r0 = (wid % CHUNKS_PER_B) * ROWS
    lanes = lax.iota(jnp.int32, L)
    rots = [((lanes + r) % L) * CPAD for r in range(4)]
    srow = lax.axis_index("s") * SROWS
    semd = [semd0, semd1, semd2]
    sems = [sems0, sems1]
    bufs = [buf0, buf1]

    def slot(k):
        return spm.at[pl.ds(srow + k * ROWS, ROWS), :]

    def d_issue(ch, k):
        pltpu.async_copy(img_hbm.at[b, ch, pl.ds(r0, ROWS), :], slot(k), semd[k])

    def d_wait(k):
        pltpu.make_async_copy(img_hbm.at[b, 0, pl.ds(r0, ROWS), :],
                              slot(k), semd[k]).wait()

    HR = ROWS // 2

    def half(k, h):
        return spm.at[pl.ds(srow + k * ROWS + h * HR, HR), :]

    def s_issue(k, v):
        pltpu.async_copy(half(k, 0), bufs[v].at[pl.ds(0, HR), :], sems[v])
        pltpu.async_copy(half(k, 1), bufs[v].at[pl.ds(HR, HR), :], sems[v])

    def s_wait(k, v):
        pltpu.make_async_copy(half(k, 0), bufs[v].at[pl.ds(0, HR), :], sems[v]).wait()
        pltpu.make_async_copy(half(k, 1), bufs[v].at[pl.ds(HR, HR), :], sems[v]).wait()

    pltpu.sync_copy(gt_hbm.at[b, pl.ds(r0, ROWS), :], gt_v)
    d_issue(0, 0)
    d_issue(1, 1)
    d_issue(2, 2)

    _zero_acc(acc)
    ones = jnp.ones((L,), jnp.float32)

    def cbody(r, carry):
        for k in range(GROUPS_PER_ROW):
            idx = rots[k % 4] + gt_v[r, pl.ds(k * L, L)]
            plsc.addupdate_scatter(acc, [idx], ones)
        return carry

    lax.fori_loop(0, ROWS, cbody, 0)
    lo, hi = _fold_acc(acc)
    cslab[pl.ds(0, L)] = lo
    cslab[pl.ds(L, L)] = hi
    pltpu.sync_copy(cslab, cnts_out.at[pl.ds(wid * CPAD, CPAD)])
    _zero_acc(acc)

    d_wait(0)
    s_issue(0, 0)

    def process(ch, v):
        buf = bufs[v]

        def rbody(r, carry):
            for k in range(GROUPS_PER_ROW):
                idx = rots[k % 4] + gt_v[r, pl.ds(k * L, L)]
                plsc.addupdate_scatter(acc, [idx], buf[r, pl.ds(k * L, L)])
            return carry

        lax.fori_loop(0, ROWS, rbody, 0)
        slo, shi = _fold_acc(acc)
        slab[pl.ds(ch * CPAD, L)] = slo
        slab[pl.ds(ch * CPAD + L, L)] = shi
        _zero_acc(acc)

    def qbody(q, carry):
        for j in range(6):
            ch = 6 * q + j
            sk, v = j % NSLOT, j % 2
            s_wait(sk, v)

            @pl.when(ch + 3 < C)
            def _refill():
                d_issue(ch + 3, sk)

            @pl.when(ch + 1 < C)
            def _stage_next():
                d_wait((sk + 1) % NSLOT)
                s_issue((sk + 1) % NSLOT, 1 - v)

        return carry

    lax.fori_loop(0, C // 6, qbody, 0)
    pltpu.sync_copy(slab, sums_out.at[pl.ds(wid * SLAB, SLAB)])


@functools.partial(
    pl.kernel,
    out_type=jax.ShapeDtypeStruct((B, C, H, W), jnp.float32),
    mesh=_mesh,
    scratch_types=[
        pltpu.VMEM((ROWS, W), jnp.int32),
        pltpu.VMEM((ROWS, W), jnp.float32),
        pltpu.VMEM((ROWS, W), jnp.float32),
        pltpu.VMEM((SLAB,), jnp.float32),
        pltpu.VMEM((SLAB,), jnp.float32),
        pltpu.VMEM((CPAD,), jnp.float32),
        pltpu.VMEM((CPAD,), jnp.float32),
        pltpu.VMEM_SHARED((NS * SROWS, W), jnp.float32),
        pltpu.SemaphoreType.DMA,
        pltpu.SemaphoreType.DMA,
        pltpu.SemaphoreType.DMA,
        pltpu.SemaphoreType.DMA,
        pltpu.SemaphoreType.DMA,
        pltpu.SemaphoreType.DMA,
    ],
    compiler_params=_params,
)
def _apply(sums_hbm, cnts_hbm, gt_hbm, out_hbm,
           gt_v, buf0, buf1, mtab, tmp, cacc, ctmp, spm,
           semd0, semd1, semd2, sems0, sems1, semg):
    wid = lax.axis_index("s") * NC + lax.axis_index("c")
    b = wid // CHUNKS_PER_B
    r0 = (wid % CHUNKS_PER_B) * ROWS
    w0 = b * CHUNKS_PER_B
    srow = lax.axis_index("s") * SROWS
    semd = [semd0, semd1, semd2]
    sems = [sems0, sems1]
    bufs = [buf0, buf1]

    def slot(k):
        return spm.at[pl.ds(srow + k * ROWS, ROWS), :]

    def d_issue(ch, k):
        pltpu.async_copy(slot(k), out_hbm.at[b, ch, pl.ds(r0, ROWS), :], semd[k])

    def d_wait(k):
        pltpu.make_async_copy(slot(k), out_hbm.at[b, 0, pl.ds(r0, ROWS), :],
                              semd[k]).wait()

    HR = ROWS // 2

    def half(k, h):
        return spm.at[pl.ds(srow + k * ROWS + h * HR, HR), :]

    def s_issue(v, k):
        pltpu.async_copy(bufs[v].at[pl.ds(0, HR), :], half(k, 0), sems[v])
        pltpu.async_copy(bufs[v].at[pl.ds(HR, HR), :], half(k, 1), sems[v])

    def s_wait(v, k):
        pltpu.make_async_copy(bufs[v].at[pl.ds(0, HR), :], half(k, 0), sems[v]).wait()
        pltpu.make_async_copy(bufs[v].at[pl.ds(HR, HR), :], half(k, 1), sems[v]).wait()

    pltpu.async_copy(gt_hbm.at[b, pl.ds(r0, ROWS), :], gt_v, semg)

    pltpu.sync_copy(sums_hbm.at[pl.ds(w0 * SLAB, SLAB)], mtab)
    pltpu.sync_copy(cnts_hbm.at[pl.ds(w0 * CPAD, CPAD)], cacc)
    for j in range(1, CHUNKS_PER_B):
        pltpu.sync_copy(sums_hbm.at[pl.ds((w0 + j) * SLAB, SLAB)], tmp)
        pltpu.sync_copy(cnts_hbm.at[pl.ds((w0 + j) * CPAD, CPAD)], ctmp)
        cacc[pl.ds(0, L)] = cacc[pl.ds(0, L)] + ctmp[pl.ds(0, L)]
        cacc[pl.ds(L, L)] = cacc[pl.ds(L, L)] + ctmp[pl.ds(L, L)]

        def abody(r, carry):
            o = r * CPAD
            mtab[pl.ds(o, L)] = mtab[pl.ds(o, L)] + tmp[pl.ds(o, L)]
            mtab[pl.ds(o + L, L)] = mtab[pl.ds(o + L, L)] + tmp[pl.ds(o + L, L)]
            return carry

        lax.fori_loop(0, C, abody, 0)

    rlo = 1.0 / (cacc[pl.ds(0, L)] + 1e-8)
    rhi = 1.0 / (cacc[pl.ds(L, L)] + 1e-8)

    def mbody(r, carry):
        o = r * CPAD
        mtab[pl.ds(o, L)] = mtab[pl.ds(o, L)] * rlo
        mtab[pl.ds(o + L, L)] = mtab[pl.ds(o + L, L)] * rhi
        return carry

    lax.fori_loop(0, C, mbody, 0)

    pltpu.make_async_copy(gt_hbm.at[b, pl.ds(r0, ROWS), :], gt_v, semg).wait()

    def process(ch, v):
        buf = bufs[v]
        cv = jnp.full((L,), ch * CPAD, dtype=jnp.int32)

        def rbody(r, carry):
            for k in range(GROUPS_PER_ROW):
                g = gt_v[r, pl.ds(k * L, L)]
                buf[r, pl.ds(k * L, L)] = plsc.load_gather(mtab, [cv + g])
            return carry

        lax.fori_loop(0, ROWS, rbody, 0)

    def qbody(q, carry):
        for j in range(6):
            ch = 6 * q + j
            sk, v = j % NSLOT, j % 2

            @pl.when(ch >= NSLOT)
            def _slot_free():
                d_wait(sk)

            process(ch, v)

            s_issue(v, sk)

            @pl.when(ch >= 1)
            def _push_prev():
                s_wait(1 - v, (sk + NSLOT - 1) % NSLOT)
                d_issue(ch - 1, (sk + NSLOT - 1) % NSLOT)
        return carry

    lax.fori_loop(0, C // 6, qbody, 0)

    s_wait(1, 2)
    d_issue(C - 1, 2)
    d_wait(0)
    d_wait(1)
    d_wait(2)


def kernel(img, gt):
    gt3 = gt.reshape(B, H, W).astype(jnp.int32)
    sums, cnts = _seg_sums(img, gt3)
    return _apply(sums, cnts, gt3)


# speedup vs baseline: 2.6375x; 1.1739x over previous
import functools

import jax
import jax.numpy as jnp
from jax import lax
from jax.experimental import pallas as pl
from jax.experimental.pallas import tpu as pltpu
from jax.experimental.pallas import tpu_sc as plsc

B, C, H, W = 4, 96, 384, 384
HW = H * W
CLS = 19
CPAD = 32
NC, NS, L = 2, 16, 16
NW = NC * NS
CHUNKS_PER_B = NW // B
ROWS = H // CHUNKS_PER_B
GROUPS_PER_ROW = W // L
SLAB = C * CPAD
NSLOT = 3
SROWS = NSLOT * ROWS

_mesh = plsc.VectorSubcoreMesh(core_axis_name="c", subcore_axis_name="s")
_params = pltpu.CompilerParams(needs_layout_passes=False, use_tc_tiling_on_sc=True)


def _zero_acc(acc):
    z = jnp.zeros((L,), jnp.float32)
    for r in range(2 * L):
        acc[pl.ds(r * L, L)] = z


def _fold_acc(acc):
    lo = acc[pl.ds(0, L)]
    hi = acc[pl.ds(L, L)]
    for r in range(1, L):
        lo = lo + acc[pl.ds(r * CPAD, L)]
        hi = hi + acc[pl.ds(r * CPAD + L, L)]
    return lo, hi


@functools.partial(
    pl.kernel,
    out_type=(
        jax.ShapeDtypeStruct((NW * SLAB,), jnp.float32),
        jax.ShapeDtypeStruct((NW * CPAD,), jnp.float32),
    ),
    mesh=_mesh,
    scratch_types=[
        pltpu.VMEM((ROWS, W), jnp.int32),
        pltpu.VMEM((ROWS, W), jnp.float32),
        pltpu.VMEM((ROWS, W), jnp.float32),
        pltpu.VMEM((L * CPAD,), jnp.float32),
        pltpu.VMEM((SLAB,), jnp.float32),
        pltpu.VMEM((CPAD,), jnp.float32),
        pltpu.VMEM_SHARED((NS * SROWS, W), jnp.float32),
        pltpu.SemaphoreType.DMA,
        pltpu.SemaphoreType.DMA,
        pltpu.SemaphoreType.DMA,
        pltpu.SemaphoreType.DMA,
        pltpu.SemaphoreType.DMA,
    ],
    compiler_params=_params,
)
def _seg_sums(img_hbm, gt_hbm, sums_out, cnts_out,
              gt_v, buf0, buf1, acc, slab, cslab, spm,
              semd0, semd1, semd2, sems0, sems1):
    wid = lax.axis_index("s") * NC + lax.axis_index("c")
    b = wid // CHUNKS_PER_B
    r0 = (wid % CHUNKS_PER_B) * ROWS
    lanes = lax.iota(jnp.int32, L)
    rots = [((lanes + r) % L) * CPAD for r in range(4)]
    srow = lax.axis_index("s") * SROWS
    semd = [semd0, semd1, semd2]
    sems = [sems0, sems1]
    bufs = [buf0, buf1]

    def slot(k):
        return spm.at[pl.ds(srow + k * ROWS, ROWS), :]

    def d_issue(ch, k):
        pltpu.async_copy(img_hbm.at[b, ch, pl.ds(r0, ROWS), :], slot(k), semd[k])

    def d_wait(k):
        pltpu.make_async_copy(img_hbm.at[b, 0, pl.ds(r0, ROWS), :],
                              slot(k), semd[k]).wait()

    HR = ROWS // 2

    def half(k, h):
        return spm.at[pl.ds(srow + k * ROWS + h * HR, HR), :]

    def s_issue(k, v):
        pltpu.async_copy(half(k, 0), bufs[v].at[pl.ds(0, HR), :], sems[v])
        pltpu.async_copy(half(k, 1), bufs[v].at[pl.ds(HR, HR), :], sems[v])

    def s_wait(k, v):
        pltpu.make_async_copy(half(k, 0), bufs[v].at[pl.ds(0, HR), :], sems[v]).wait()
        pltpu.make_async_copy(half(k, 1), bufs[v].at[pl.ds(HR, HR), :], sems[v]).wait()

    pltpu.sync_copy(gt_hbm.at[b, pl.ds(r0, ROWS), :], gt_v)
    d_issue(0, 0)
    d_issue(1, 1)
    d_issue(2, 2)

    _zero_acc(acc)
    ones = jnp.ones((L,), jnp.float32)

    @plsc.parallel_loop(0, ROWS, unroll=2)
    def cbody(r):
        for k in range(GROUPS_PER_ROW):
            idx = rots[k % 4] + gt_v[r, pl.ds(k * L, L)]
            plsc.addupdate_scatter(acc, [idx], ones)

    lo, hi = _fold_acc(acc)
    cslab[pl.ds(0, L)] = lo
    cslab[pl.ds(L, L)] = hi
    pltpu.sync_copy(cslab, cnts_out.at[pl.ds(wid * CPAD, CPAD)])
    _zero_acc(acc)

    d_wait(0)
    s_issue(0, 0)

    def process(ch, v):
        buf = bufs[v]

        @plsc.parallel_loop(0, ROWS, unroll=2)
        def rbody(r):
            for k in range(GROUPS_PER_ROW):
                idx = rots[k % 4] + gt_v[r, pl.ds(k * L, L)]
                plsc.addupdate_scatter(acc, [idx], buf[r, pl.ds(k * L, L)])

        slo, shi = _fold_acc(acc)
        slab[pl.ds(ch * CPAD, L)] = slo
        slab[pl.ds(ch * CPAD + L, L)] = shi
        _zero_acc(acc)

    def qbody(q, carry):
        for j in range(6):
            ch = 6 * q + j
            sk, v = j % NSLOT, j % 2
            s_wait(sk, v)

            @pl.when(ch + 3 < C)
            def _refill():
                d_issue(ch + 3, sk)

            @pl.when(ch + 1 < C)
            def _stage_next():
                d_wait((sk + 1) % NSLOT)
                s_issue((sk + 1) % NSLOT, 1 - v)

            process(ch, v)
        return carry

    lax.fori_loop(0, C // 6, qbody, 0)
    pltpu.sync_copy(slab, sums_out.at[pl.ds(wid * SLAB, SLAB)])


@functools.partial(
    pl.kernel,
    out_type=jax.ShapeDtypeStruct((B, C, H, W), jnp.float32),
    mesh=_mesh,
    scratch_types=[
        pltpu.VMEM((ROWS, W), jnp.int32),
        pltpu.VMEM((ROWS, W), jnp.float32),
        pltpu.VMEM((ROWS, W), jnp.float32),
        pltpu.VMEM((SLAB,), jnp.float32),
        pltpu.VMEM((SLAB,), jnp.float32),
        pltpu.VMEM((CPAD,), jnp.float32),
        pltpu.VMEM((CPAD,), jnp.float32),
        pltpu.VMEM_SHARED((NS * SROWS, W), jnp.float32),
        pltpu.SemaphoreType.DMA,
        pltpu.SemaphoreType.DMA,
        pltpu.SemaphoreType.DMA,
        pltpu.SemaphoreType.DMA,
        pltpu.SemaphoreType.DMA,
        pltpu.SemaphoreType.DMA,
    ],
    compiler_params=_params,
)
def _apply(sums_hbm, cnts_hbm, gt_hbm, out_hbm,
           gt_v, buf0, buf1, mtab, tmp, cacc, ctmp, spm,
           semd0, semd1, semd2, sems0, sems1, semg):
    wid = lax.axis_index("s") * NC + lax.axis_index("c")
    b = wid // CHUNKS_PER_B
    r0 = (wid % CHUNKS_PER_B) * ROWS
    w0 = b * CHUNKS_PER_B
    srow = lax.axis_index("s") * SROWS
    semd = [semd0, semd1, semd2]
    sems = [sems0, sems1]
    bufs = [buf0, buf1]

    def slot(k):
        return spm.at[pl.ds(srow + k * ROWS, ROWS), :]

    def d_issue(ch, k):
        pltpu.async_copy(slot(k), out_hbm.at[b, ch, pl.ds(r0, ROWS), :], semd[k])

    def d_wait(k):
        pltpu.make_async_copy(slot(k), out_hbm.at[b, 0, pl.ds(r0, ROWS), :],
                              semd[k]).wait()

    HR = ROWS // 2

    def half(k, h):
        return spm.at[pl.ds(srow + k * ROWS + h * HR, HR), :]

    def s_issue(v, k):
        pltpu.async_copy(bufs[v].at[pl.ds(0, HR), :], half(k, 0), sems[v])
        pltpu.async_copy(bufs[v].at[pl.ds(HR, HR), :], half(k, 1), sems[v])

    def s_wait(v, k):
        pltpu.make_async_copy(bufs[v].at[pl.ds(0, HR), :], half(k, 0), sems[v]).wait()
        pltpu.make_async_copy(bufs[v].at[pl.ds(HR, HR), :], half(k, 1), sems[v]).wait()

    pltpu.async_copy(gt_hbm.at[b, pl.ds(r0, ROWS), :], gt_v, semg)

    pltpu.sync_copy(sums_hbm.at[pl.ds(w0 * SLAB, SLAB)], mtab)
    pltpu.sync_copy(cnts_hbm.at[pl.ds(w0 * CPAD, CPAD)], cacc)
    for j in range(1, CHUNKS_PER_B):
        pltpu.sync_copy(sums_hbm.at[pl.ds((w0 + j) * SLAB, SLAB)], tmp)
        pltpu.sync_copy(cnts_hbm.at[pl.ds((w0 + j) * CPAD, CPAD)], ctmp)
        cacc[pl.ds(0, L)] = cacc[pl.ds(0, L)] + ctmp[pl.ds(0, L)]
        cacc[pl.ds(L, L)] = cacc[pl.ds(L, L)] + ctmp[pl.ds(L, L)]

        def abody(r, carry):
            o = r * CPAD
            mtab[pl.ds(o, L)] = mtab[pl.ds(o, L)] + tmp[pl.ds(o, L)]
            mtab[pl.ds(o + L, L)] = mtab[pl.ds(o + L, L)] + tmp[pl.ds(o + L, L)]
            return carry

        lax.fori_loop(0, C, abody, 0)

    rlo = 1.0 / (cacc[pl.ds(0, L)] + 1e-8)
    rhi = 1.0 / (cacc[pl.ds(L, L)] + 1e-8)

    def mbody(r, carry):
        o = r * CPAD
        mtab[pl.ds(o, L)] = mtab[pl.ds(o, L)] * rlo
        mtab[pl.ds(o + L, L)] = mtab[pl.ds(o + L, L)] * rhi
        return carry

    lax.fori_loop(0, C, mbody, 0)

    pltpu.make_async_copy(gt_hbm.at[b, pl.ds(r0, ROWS), :], gt_v, semg).wait()

    def process(ch, v):
        buf = bufs[v]
        cv = jnp.full((L,), ch * CPAD, dtype=jnp.int32)

        @plsc.parallel_loop(0, ROWS, unroll=2)
        def rbody(r):
            for k in range(GROUPS_PER_ROW):
                g = gt_v[r, pl.ds(k * L, L)]
                buf[r, pl.ds(k * L, L)] = plsc.load_gather(mtab, [cv + g])


    def qbody(q, carry):
        for j in range(6):
            ch = 6 * q + j
            sk, v = j % NSLOT, j % 2

            @pl.when(ch >= NSLOT)
            def _slot_free():
                d_wait(sk)

            process(ch, v)

            s_issue(v, sk)

            @pl.when(ch >= 1)
            def _push_prev():
                s_wait(1 - v, (sk + NSLOT - 1) % NSLOT)
                d_issue(ch - 1, (sk + NSLOT - 1) % NSLOT)
        return carry

    lax.fori_loop(0, C // 6, qbody, 0)

    s_wait(1, 2)
    d_issue(C - 1, 2)
    d_wait(0)
    d_wait(1)
    d_wait(2)


def kernel(img, gt):
    gt3 = gt.reshape(B, H, W).astype(jnp.int32)
    sums, cnts = _seg_sums(img, gt3)
    return _apply(sums, cnts, gt3)
